# baseline (speedup 1.0000x reference)
import functools

import jax
import jax.numpy as jnp
from jax import lax
from jax.experimental import pallas as pl
from jax.experimental.pallas import tpu as pltpu
from jax.experimental.pallas import tpu_sc as plsc

V = 9217
D = 192
DA = 128
DB = D - DA
NC = 2
NS = 16
NW = NC * NS
CHUNK = 128
NBUF = 2


@functools.partial(jax.jit, static_argnames=("batch", "seq"))
def _lookup(ids_flat, table, *, batch, seq):
    B = batch * seq
    per_w = B // NW
    G = per_w // CHUNK

    ids3 = ids_flat.reshape(NW, G, CHUNK)
    table_a = table[:, :DA]
    table_b = jnp.pad(table[:, DA:], ((0, 0), (0, DA - DB)))

    mesh = plsc.VectorSubcoreMesh(core_axis_name="c", subcore_axis_name="s")

    @functools.partial(
        pl.kernel,
        out_type=jax.ShapeDtypeStruct((B, D), jnp.float32),
        mesh=mesh,
        scratch_types=[
            pltpu.VMEM((G, CHUNK), jnp.int32),
            [pltpu.VMEM((CHUNK, DA), jnp.float32) for _ in range(NBUF)],
            [pltpu.VMEM((CHUNK, DA), jnp.float32) for _ in range(NBUF)],
            [pltpu.VMEM((CHUNK, DB), jnp.float32) for _ in range(NBUF)],
            [pltpu.SemaphoreType.DMA for _ in range(NBUF)],
            [pltpu.SemaphoreType.DMA for _ in range(NBUF)],
            [pltpu.SemaphoreType.DMA for _ in range(NBUF)],
            [pltpu.SemaphoreType.DMA for _ in range(NBUF)],
        ],
    )
    def k(ids_hbm, ta_hbm, tb_hbm, out_hbm, idx_v, bufa, bufb, bufb64,
          gsa, gsb, wsa, wsb):
        wid = lax.axis_index("s") * NC + lax.axis_index("c")
        base = wid * per_w

        pltpu.sync_copy(ids_hbm.at[wid], idx_v)

        def start_gather(g, b):
            pltpu.async_copy(ta_hbm.at[idx_v.at[g]], bufa[b], gsa[b])
            pltpu.async_copy(tb_hbm.at[idx_v.at[g]], bufb[b], gsb[b])

        def wait_gather(g, b):
            pltpu.make_async_copy(ta_hbm.at[idx_v.at[g]], bufa[b], gsa[b]).wait()
            pltpu.make_async_copy(tb_hbm.at[idx_v.at[g]], bufb[b], gsb[b]).wait()

        def compact_b(b):
            @pl.loop(0, CHUNK)
            def _(r):
                for c in range(DB // 16):
                    bufb64[b][r, pl.ds(c * 16, 16)] = (
                        bufb[b][r, pl.ds(c * 16, 16)]
                    )

        def start_write_a(g, b):
            r0 = base + g * CHUNK
            pltpu.async_copy(
                bufa[b], out_hbm.at[pl.ds(r0, CHUNK), pl.ds(0, DA)], wsa[b]
            )

        def start_write_b(g, b):
            r0 = base + g * CHUNK
            pltpu.async_copy(
                bufb64[b],
                out_hbm.at[pl.ds(r0, CHUNK), pl.ds(DA, DB)],
                wsb[b],
            )

        def wait_write(g, b):
            r0 = base + g * CHUNK
            pltpu.make_async_copy(
                bufa[b], out_hbm.at[pl.ds(r0, CHUNK), pl.ds(0, DA)], wsa[b]
            ).wait()
            pltpu.make_async_copy(
                bufb64[b],
                out_hbm.at[pl.ds(r0, CHUNK), pl.ds(DA, DB)],
                wsb[b],
            ).wait()

        start_gather(0, 0)

        @pl.loop(0, G, step=NBUF)
        def _(g0):
            for j in range(NBUF):
                g = g0 + j
                b = j
                wait_gather(g, b)
                b2 = (j + 1) % NBUF

                @pl.when(g >= 1)
                def _():
                    wait_write(g - 1, b2)

                @pl.when(g + 1 < G)
                def _():
                    start_gather(g + 1, b2)

                start_write_a(g, b)
                compact_b(b)
                start_write_b(g, b)

        wait_write(G - 1, (G - 1) % NBUF)

    return k(ids3, table_a, table_b)


def kernel(position_ids, table):
    batch, seq = position_ids.shape
    ids_flat = position_ids.reshape(-1).astype(jnp.int32)
    out = _lookup(ids_flat, table, batch=batch, seq=seq)
    return out.reshape(batch, seq, D)
